# baseline (speedup 1.0000x reference)
import jax
import jax.numpy as jnp
from jax import lax
from jax.experimental import pallas as pl
from jax.experimental.pallas import tpu as pltpu
from jax.experimental.pallas import tpu_sc as plsc

N = 1_000_000
B = 4096
K = 64
F = 64
L = 16
NC = 2
NS = 16
NW = NC * NS
BW = B // NW
CB = 4
CI = CB * K
NCH = BW // CB
GP = CI // L
JB = F // L


def _rsqrt(x):
    xi = plsc.bitcast(x, jnp.int32)
    xi = jnp.int32(0x5F3759DF) - (xi >> 1)
    r = plsc.bitcast(xi, jnp.float32)
    for _ in range(3):
        r = r * (1.5 - 0.5 * x * r * r)
    return r


def _sc_body(feat_hbm, y_hbm, idx_hbm, upd_hbm, bank_hbm, out_hbm,
             marker, nfsh, y_v, bv_v, u_v, fbuf, nfstage, st, rs_v,
             ix_v, m_v, rbuf, rfix, obuf, fb2, gsem, msem, osem):
    c = lax.axis_index("c")
    s = lax.axis_index("s")
    wid = s * NC + c
    iota = lax.iota(jnp.int32, L)

    pltpu.sync_copy(y_hbm, y_v)
    pltpu.sync_copy(upd_hbm, u_v)

    def nf_group(g, carry):
        b0 = s * (B // NS) + g * L
        pltpu.sync_copy(feat_hbm.at[pl.ds(b0, L)], fbuf)
        for i in range(L):
            acc = None
            for j in range(JB):
                v = fbuf[i, pl.ds(j * L, L)]
                acc = v * v if acc is None else acc + v * v
            plsc.store_scatter(st.at[0], [iota * 17 + i], acc)
        nrm2 = st[0, pl.ds(0, L)]
        for l in range(1, L):
            nrm2 = nrm2 + st[0, pl.ds(l * 17, L)]
        rs_v[...] = _rsqrt(nrm2)
        for i in range(L):
            sc = plsc.load_gather(rs_v, [jnp.full((L,), i, jnp.int32)])
            for j in range(JB):
                nfstage[i, pl.ds(j * L, L)] = fbuf[i, pl.ds(j * L, L)] * sc
        pltpu.sync_copy(nfstage, nfsh.at[pl.ds(b0, L)])
        return carry

    lax.fori_loop(0, (B // NS) // L, nf_group, 0)

    @pl.when(s == 0)
    def _scatter_marker():
        def bv_fill(g, carry):
            bv_v[pl.ds(g * L, L)] = iota + g * L
            return carry
        lax.fori_loop(0, B // L, bv_fill, 0)
        pltpu.sync_copy(bv_v, marker.at[y_v])

    plsc.subcore_barrier()

    u = u_v[...]

    def issue(ci):
        p = lax.rem(ci, 2)
        b0 = wid * BW + ci * CB
        e0 = b0 * K
        pltpu.sync_copy(idx_hbm.at[pl.ds(e0, CI)], ix_v.at[p])
        pltpu.async_copy(bank_hbm.at[ix_v.at[p]], rbuf.at[p], gsem.at[p])
        pltpu.async_copy(marker.at[ix_v.at[p]], m_v.at[p], msem.at[p])
        pltpu.sync_copy(feat_hbm.at[pl.ds(b0, CB)], fb2.at[p])

    issue(0)

    def chunk(ci, carry):
        p = lax.rem(ci, 2)
        b0 = wid * BW + ci * CB
        e0 = b0 * K

        @pl.when(ci + 1 < NCH)
        def _issue_next():
            issue(ci + 1)

        pltpu.make_async_copy(bank_hbm.at[ix_v.at[p]], rbuf.at[p],
                              gsem.at[p]).wait()
        pltpu.make_async_copy(marker.at[ix_v.at[p]], m_v.at[p],
                              msem.at[p]).wait()

        @pl.when(ci >= 2)
        def _drain_out():
            pltpu.make_async_copy(obuf.at[p], out_hbm.at[pl.ds(e0, CI)],
                                  osem.at[p]).wait()

        def pairg(h, hcarry):
            for q in range(2):
                g = h * 2 + q
                boff = g // (K // L)
                rb = g * L
                fb = [fb2[p, boff, pl.ds(j * L, L)] for j in range(JB)]

                def dots(rows):
                    for i in range(L):
                        acc = None
                        for j in range(JB):
                            v = rows[i, pl.ds(j * L, L)] * fb[j]
                            acc = v if acc is None else acc + v
                        plsc.store_scatter(st.at[q], [iota * 17 + i], acc)
                    tot = st[q, pl.ds(0, L)]
                    for l in range(1, L):
                        tot = tot + st[q, pl.ds(l * 17, L)]
                    return tot

                base = dots(rbuf.at[p, pl.ds(rb, L)])
                m = m_v[p, pl.ds(rb, L)]
                r = ix_v[p, pl.ds(rb, L)]
                mc = jnp.clip(m, 0, B - 1)
                yv = plsc.load_gather(y_v, [mc])
                valid = (m >= 0) & (m < B) & (yv == r) & (u != 0)
                obuf[p, pl.ds(rb, L)] = base
                nv = jnp.sum(valid.astype(jnp.int32))

                @pl.when(nv > 0)
                def _fix():
                    pltpu.sync_copy(nfsh.at[mc], rfix)
                    fix = dots(rfix)
                    obuf[p, pl.ds(rb, L)] = jnp.where(valid, fix, base)

            return hcarry

        lax.fori_loop(0, GP // 2, pairg, 0)
        pltpu.async_copy(obuf.at[p], out_hbm.at[pl.ds(e0, CI)], osem.at[p])
        return carry

    lax.fori_loop(0, NCH, chunk, 0)

    for q in (NCH - 2, NCH - 1):
        pq = q % 2
        eq = (wid * BW + q * CB) * K
        pltpu.make_async_copy(obuf.at[pq], out_hbm.at[pl.ds(eq, CI)],
                              osem.at[pq]).wait()


@jax.jit
def kernel(feature, y, idx, update, memory_bank):
    mesh = plsc.VectorSubcoreMesh(core_axis_name="c", subcore_axis_name="s")
    run = pl.kernel(
        _sc_body,
        out_type=jax.ShapeDtypeStruct((B * K,), jnp.float32),
        mesh=mesh,
        compiler_params=pltpu.CompilerParams(needs_layout_passes=False,
                                              use_tc_tiling_on_sc=False),
        scratch_types=[
            pltpu.VMEM_SHARED((N,), jnp.int32),
            pltpu.VMEM_SHARED((B, F), jnp.float32),
            pltpu.VMEM((B,), jnp.int32),
            pltpu.VMEM((B,), jnp.int32),
            pltpu.VMEM((L,), jnp.int32),
            pltpu.VMEM((L, F), jnp.float32),
            pltpu.VMEM((L, F), jnp.float32),
            pltpu.VMEM((2, L * 17), jnp.float32),
            pltpu.VMEM((L,), jnp.float32),
            pltpu.VMEM((2, CI), jnp.int32),
            pltpu.VMEM((2, CI), jnp.int32),
            pltpu.VMEM((2, CI, F), jnp.float32),
            pltpu.VMEM((L, F), jnp.float32),
            pltpu.VMEM((2, CI), jnp.float32),
            pltpu.VMEM((2, CB, F), jnp.float32),
            pltpu.SemaphoreType.DMA((2,)),
            pltpu.SemaphoreType.DMA((2,)),
            pltpu.SemaphoreType.DMA((2,)),
        ],
    )
    upd_vec = jnp.full((L,), update, jnp.int32)
    out = run(feature, y, idx.reshape(-1), upd_vec, memory_bank)
    return out.reshape(B, K, 1)


# speedup vs baseline: 1.0013x; 1.0013x over previous
import jax
import jax.numpy as jnp
from jax import lax
from jax.experimental import pallas as pl
from jax.experimental.pallas import tpu as pltpu
from jax.experimental.pallas import tpu_sc as plsc

N = 1_000_000
B = 4096
K = 64
F = 64
L = 16
NC = 2
NS = 16
NW = NC * NS
BW = B // NW
CB = 4
CI = CB * K
NCH = BW // CB
GP = CI // L
JB = F // L


def _rsqrt(x):
    xi = plsc.bitcast(x, jnp.int32)
    xi = jnp.int32(0x5F3759DF) - (xi >> 1)
    r = plsc.bitcast(xi, jnp.float32)
    for _ in range(3):
        r = r * (1.5 - 0.5 * x * r * r)
    return r


def _sc_body(feat_hbm, y_hbm, idx_hbm, upd_hbm, bank_hbm, out_hbm,
             marker, nfsh, y_v, bv_v, u_v, fbuf, nfstage, st, rs_v,
             ix_v, m_v, rbuf, rfix, obuf, fb2, gsem, msem, osem):
    c = lax.axis_index("c")
    s = lax.axis_index("s")
    wid = s * NC + c
    iota = lax.iota(jnp.int32, L)

    pltpu.sync_copy(y_hbm, y_v)
    pltpu.sync_copy(upd_hbm, u_v)

    def nf_group(g, carry):
        b0 = s * (B // NS) + g * L
        pltpu.sync_copy(feat_hbm.at[pl.ds(b0, L)], fbuf)
        for i in range(L):
            acc = None
            for j in range(JB):
                v = fbuf[i, pl.ds(j * L, L)]
                acc = v * v if acc is None else acc + v * v
            plsc.store_scatter(st.at[0], [iota * 17 + i], acc)
        nrm2 = st[0, pl.ds(0, L)]
        for l in range(1, L):
            nrm2 = nrm2 + st[0, pl.ds(l * 17, L)]
        rs_v[...] = _rsqrt(nrm2)
        for i in range(L):
            sc = plsc.load_gather(rs_v, [jnp.full((L,), i, jnp.int32)])
            for j in range(JB):
                nfstage[i, pl.ds(j * L, L)] = fbuf[i, pl.ds(j * L, L)] * sc
        pltpu.sync_copy(nfstage, nfsh.at[pl.ds(b0, L)])
        return carry

    with jax.named_scope("phase0"):
        lax.fori_loop(0, (B // NS) // L, nf_group, 0)

    @pl.when(s == 0)
    def _scatter_marker():
        def bv_fill(g, carry):
            bv_v[pl.ds(g * L, L)] = iota + g * L
            return carry
        lax.fori_loop(0, B // L, bv_fill, 0)
        pltpu.sync_copy(bv_v, marker.at[y_v])

    plsc.subcore_barrier()

    u = u_v[...]

    def issue(ci):
        p = lax.rem(ci, 2)
        b0 = wid * BW + ci * CB
        e0 = b0 * K
        pltpu.sync_copy(idx_hbm.at[pl.ds(e0, CI)], ix_v.at[p])
        pltpu.async_copy(bank_hbm.at[ix_v.at[p]], rbuf.at[p], gsem.at[p])
        pltpu.async_copy(marker.at[ix_v.at[p]], m_v.at[p], msem.at[p])
        pltpu.sync_copy(feat_hbm.at[pl.ds(b0, CB)], fb2.at[p])

    issue(0)

    def chunk(ci, carry):
        p = lax.rem(ci, 2)
        b0 = wid * BW + ci * CB
        e0 = b0 * K

        @pl.when(ci + 1 < NCH)
        def _issue_next():
            issue(ci + 1)

        with jax.named_scope("dmawait"):
            pltpu.make_async_copy(bank_hbm.at[ix_v.at[p]], rbuf.at[p],
                                  gsem.at[p]).wait()
            pltpu.make_async_copy(marker.at[ix_v.at[p]], m_v.at[p],
                                  msem.at[p]).wait()

        @pl.when(ci >= 2)
        def _drain_out():
            pltpu.make_async_copy(obuf.at[p], out_hbm.at[pl.ds(e0, CI)],
                                  osem.at[p]).wait()

        def pairg(h, hcarry):
            for q in range(2):
                g = h * 2 + q
                boff = g // (K // L)
                rb = g * L
                fb = [fb2[p, boff, pl.ds(j * L, L)] for j in range(JB)]

                def dots(rows):
                    for i in range(L):
                        acc = None
                        for j in range(JB):
                            v = rows[i, pl.ds(j * L, L)] * fb[j]
                            acc = v if acc is None else acc + v
                        plsc.store_scatter(st.at[q], [iota * 17 + i], acc)
                    tot = st[q, pl.ds(0, L)]
                    for l in range(1, L):
                        tot = tot + st[q, pl.ds(l * 17, L)]
                    return tot

                base = dots(rbuf.at[p, pl.ds(rb, L)])
                m = m_v[p, pl.ds(rb, L)]
                r = ix_v[p, pl.ds(rb, L)]
                mc = jnp.clip(m, 0, B - 1)
                yv = plsc.load_gather(y_v, [mc])
                valid = (m >= 0) & (m < B) & (yv == r) & (u != 0)
                obuf[p, pl.ds(rb, L)] = base
                nv = jnp.sum(valid.astype(jnp.int32))

                @pl.when(nv > 0)
                def _fix():
                    pltpu.sync_copy(nfsh.at[mc], rfix)
                    fix = dots(rfix)
                    obuf[p, pl.ds(rb, L)] = jnp.where(valid, fix, base)

            return hcarry

        with jax.named_scope("dots"):
            lax.fori_loop(0, GP // 2, pairg, 0)
        pltpu.async_copy(obuf.at[p], out_hbm.at[pl.ds(e0, CI)], osem.at[p])
        return carry

    lax.fori_loop(0, NCH, chunk, 0)

    for q in (NCH - 2, NCH - 1):
        pq = q % 2
        eq = (wid * BW + q * CB) * K
        pltpu.make_async_copy(obuf.at[pq], out_hbm.at[pl.ds(eq, CI)],
                              osem.at[pq]).wait()


@jax.jit
def kernel(feature, y, idx, update, memory_bank):
    mesh = plsc.VectorSubcoreMesh(core_axis_name="c", subcore_axis_name="s")
    run = pl.kernel(
        _sc_body,
        out_type=jax.ShapeDtypeStruct((B * K,), jnp.float32),
        mesh=mesh,
        compiler_params=pltpu.CompilerParams(needs_layout_passes=False,
                                              use_tc_tiling_on_sc=False),
        scratch_types=[
            pltpu.VMEM_SHARED((N,), jnp.int32),
            pltpu.VMEM_SHARED((B, F), jnp.float32),
            pltpu.VMEM((B,), jnp.int32),
            pltpu.VMEM((B,), jnp.int32),
            pltpu.VMEM((L,), jnp.int32),
            pltpu.VMEM((L, F), jnp.float32),
            pltpu.VMEM((L, F), jnp.float32),
            pltpu.VMEM((2, L * 17), jnp.float32),
            pltpu.VMEM((L,), jnp.float32),
            pltpu.VMEM((2, CI), jnp.int32),
            pltpu.VMEM((2, CI), jnp.int32),
            pltpu.VMEM((2, CI, F), jnp.float32),
            pltpu.VMEM((L, F), jnp.float32),
            pltpu.VMEM((2, CI), jnp.float32),
            pltpu.VMEM((2, CB, F), jnp.float32),
            pltpu.SemaphoreType.DMA((2,)),
            pltpu.SemaphoreType.DMA((2,)),
            pltpu.SemaphoreType.DMA((2,)),
        ],
    )
    upd_vec = jnp.full((L,), update, jnp.int32)
    out = run(feature, y, idx.reshape(-1), upd_vec, memory_bank)
    return out.reshape(B, K, 1)


# speedup vs baseline: 1.0395x; 1.0382x over previous
import jax
import jax.numpy as jnp
from jax import lax
from jax.experimental import pallas as pl
from jax.experimental.pallas import tpu as pltpu
from jax.experimental.pallas import tpu_sc as plsc

N = 1_000_000
B = 4096
K = 64
F = 64
L = 16
NC = 2
NS = 16
NW = NC * NS
BW = B // NW
CB = 2
CI = CB * K
NCH = BW // CB
GP = CI // L
JB = F // L


def _rsqrt(x):
    xi = plsc.bitcast(x, jnp.int32)
    xi = jnp.int32(0x5F3759DF) - (xi >> 1)
    r = plsc.bitcast(xi, jnp.float32)
    for _ in range(3):
        r = r * (1.5 - 0.5 * x * r * r)
    return r


def _sc_body(feat_hbm, y_hbm, idx_hbm, upd_hbm, bank_hbm, out_hbm,
             marker, nfsh, y_v, bv_v, u_v, fbuf, nfstage, st, rs_v,
             ix_all, f_all, m_v, rbuf, rfix, obuf, gsem, msem, osem):
    c = lax.axis_index("c")
    s = lax.axis_index("s")
    wid = s * NC + c
    iota = lax.iota(jnp.int32, L)

    pltpu.sync_copy(y_hbm, y_v)
    pltpu.sync_copy(upd_hbm, u_v)
    pltpu.sync_copy(idx_hbm.at[pl.ds(wid * BW * K, BW * K)], ix_all)
    pltpu.sync_copy(feat_hbm.at[pl.ds(wid * BW, BW)], f_all)

    def nf_group(g, carry):
        b0 = s * (B // NS) + g * L
        pltpu.sync_copy(feat_hbm.at[pl.ds(b0, L)], fbuf)
        for i in range(L):
            acc = None
            for j in range(JB):
                v = fbuf[i, pl.ds(j * L, L)]
                acc = v * v if acc is None else acc + v * v
            plsc.store_scatter(st.at[0], [iota * 17 + i], acc)
        nrm2 = st[0, pl.ds(0, L)]
        for l in range(1, L):
            nrm2 = nrm2 + st[0, pl.ds(l * 17, L)]
        rs_v[...] = _rsqrt(nrm2)
        for i in range(L):
            sc = plsc.load_gather(rs_v, [jnp.full((L,), i, jnp.int32)])
            for j in range(JB):
                nfstage[i, pl.ds(j * L, L)] = fbuf[i, pl.ds(j * L, L)] * sc
        pltpu.sync_copy(nfstage, nfsh.at[pl.ds(b0, L)])
        return carry

    with jax.named_scope("phase0"):
        lax.fori_loop(0, (B // NS) // L, nf_group, 0)

    @pl.when(s == 0)
    def _scatter_marker():
        def bv_fill(g, carry):
            bv_v[pl.ds(g * L, L)] = iota + g * L
            return carry
        lax.fori_loop(0, B // L, bv_fill, 0)
        pltpu.sync_copy(bv_v, marker.at[y_v])

    plsc.subcore_barrier()

    u = u_v[...]

    def issue(ci):
        p = lax.rem(ci, 2)
        ixs = ix_all.at[pl.ds(ci * CI, CI)]
        pltpu.async_copy(bank_hbm.at[ixs], rbuf.at[p], gsem.at[p])
        pltpu.async_copy(marker.at[ixs], m_v.at[p], msem.at[p])

    issue(0)

    def chunk(ci, carry):
        p = lax.rem(ci, 2)
        b0 = wid * BW + ci * CB
        e0 = b0 * K

        @pl.when(ci + 1 < NCH)
        def _issue_next():
            issue(ci + 1)

        ixs = ix_all.at[pl.ds(ci * CI, CI)]
        with jax.named_scope("dmawait"):
            pltpu.make_async_copy(bank_hbm.at[ixs], rbuf.at[p],
                                  gsem.at[p]).wait()
            pltpu.make_async_copy(marker.at[ixs], m_v.at[p],
                                  msem.at[p]).wait()

        @pl.when(ci >= 2)
        def _drain_out():
            pltpu.make_async_copy(obuf.at[p], out_hbm.at[pl.ds(e0, CI)],
                                  osem.at[p]).wait()

        def pairg(h, hcarry):
            for q in range(2):
                g = h * 2 + q
                boff = g // (K // L)
                rb = g * L
                fb = [f_all[ci * CB + boff, pl.ds(j * L, L)]
                      for j in range(JB)]

                def dots(rows):
                    for i in range(L):
                        acc = None
                        for j in range(JB):
                            v = rows[i, pl.ds(j * L, L)] * fb[j]
                            acc = v if acc is None else acc + v
                        plsc.store_scatter(st.at[q], [iota * 17 + i], acc)
                    tot = st[q, pl.ds(0, L)]
                    for l in range(1, L):
                        tot = tot + st[q, pl.ds(l * 17, L)]
                    return tot

                base = dots(rbuf.at[p, pl.ds(rb, L)])
                m = m_v[p, pl.ds(rb, L)]
                r = ix_all[pl.ds(ci * CI + rb, L)]
                mc = jnp.clip(m, 0, B - 1)
                yv = plsc.load_gather(y_v, [mc])
                valid = (m >= 0) & (m < B) & (yv == r) & (u != 0)
                obuf[p, pl.ds(rb, L)] = base
                nv = jnp.sum(valid.astype(jnp.int32))

                @pl.when(nv > 0)
                def _fix():
                    pltpu.sync_copy(nfsh.at[mc], rfix)
                    fix = dots(rfix)
                    obuf[p, pl.ds(rb, L)] = jnp.where(valid, fix, base)

            return hcarry

        with jax.named_scope("dots"):
            lax.fori_loop(0, GP // 2, pairg, 0)
        pltpu.async_copy(obuf.at[p], out_hbm.at[pl.ds(e0, CI)], osem.at[p])
        return carry

    lax.fori_loop(0, NCH, chunk, 0)

    for q in (NCH - 2, NCH - 1):
        pq = q % 2
        eq = (wid * BW + q * CB) * K
        pltpu.make_async_copy(obuf.at[pq], out_hbm.at[pl.ds(eq, CI)],
                              osem.at[pq]).wait()


@jax.jit
def kernel(feature, y, idx, update, memory_bank):
    mesh = plsc.VectorSubcoreMesh(core_axis_name="c", subcore_axis_name="s")
    run = pl.kernel(
        _sc_body,
        out_type=jax.ShapeDtypeStruct((B * K,), jnp.float32),
        mesh=mesh,
        compiler_params=pltpu.CompilerParams(needs_layout_passes=False,
                                              use_tc_tiling_on_sc=False),
        scratch_types=[
            pltpu.VMEM_SHARED((N,), jnp.int32),
            pltpu.VMEM_SHARED((B, F), jnp.float32),
            pltpu.VMEM((B,), jnp.int32),
            pltpu.VMEM((B,), jnp.int32),
            pltpu.VMEM((L,), jnp.int32),
            pltpu.VMEM((L, F), jnp.float32),
            pltpu.VMEM((L, F), jnp.float32),
            pltpu.VMEM((2, L * 17), jnp.float32),
            pltpu.VMEM((L,), jnp.float32),
            pltpu.VMEM((BW * K,), jnp.int32),
            pltpu.VMEM((BW, F), jnp.float32),
            pltpu.VMEM((2, CI), jnp.int32),
            pltpu.VMEM((2, CI, F), jnp.float32),
            pltpu.VMEM((L, F), jnp.float32),
            pltpu.VMEM((2, CI), jnp.float32),
            pltpu.SemaphoreType.DMA((2,)),
            pltpu.SemaphoreType.DMA((2,)),
            pltpu.SemaphoreType.DMA((2,)),
        ],
    )
    upd_vec = jnp.full((L,), update, jnp.int32)
    out = run(feature, y, idx.reshape(-1), upd_vec, memory_bank)
    return out.reshape(B, K, 1)
